# baseline (speedup 1.0000x reference)
import jax
import jax.numpy as jnp
from jax import lax
from jax.experimental import pallas as pl
from jax.experimental.pallas import tpu as pltpu
from jax.experimental.pallas import tpu_sc as plsc

_N = 16384
_TABLE = 10
_NC = 2
_NS = 16
_L = 16
_NW = _NC * _NS
_CHUNK = _N // _NW


def _lookup_body(y_hbm, table_hbm, out_hbm, y_v, out_v, table_v):
    wid = lax.axis_index("s") * _NC + lax.axis_index("c")
    base = wid * _CHUNK
    pltpu.sync_copy(table_hbm, table_v)
    pltpu.sync_copy(y_hbm.at[pl.ds(base, _CHUNK)], y_v)
    for i in range(_CHUNK // _L):
        idx = y_v[pl.ds(i * _L, _L)]
        out_v[pl.ds(i * _L, _L)] = plsc.load_gather(table_v, [idx])
    pltpu.sync_copy(out_v, out_hbm.at[pl.ds(base, _CHUNK)])


def kernel(y, table):
    run = pl.kernel(
        _lookup_body,
        out_type=jax.ShapeDtypeStruct((_N,), jnp.int32),
        mesh=plsc.VectorSubcoreMesh(core_axis_name="c", subcore_axis_name="s"),
        compiler_params=pltpu.CompilerParams(needs_layout_passes=False),
        scratch_types=[
            pltpu.VMEM((_CHUNK,), jnp.int32),
            pltpu.VMEM((_CHUNK,), jnp.int32),
            pltpu.VMEM((_TABLE,), jnp.int32),
        ],
    )
    return run(y.astype(jnp.int32), table.astype(jnp.int32))


# speedup vs baseline: 1.0901x; 1.0901x over previous
import jax
import jax.numpy as jnp
from jax import lax
from jax.experimental import pallas as pl
from jax.experimental.pallas import tpu as pltpu
from jax.experimental.pallas import tpu_sc as plsc

_N = 16384
_TABLE = 10
_NC = 2
_NS = 16
_L = 16
_NW = _NS
_CHUNK = _N // _NW


def _lookup_body(y_hbm, table_hbm, out_hbm, y_v, out_v, table_v, sem_t, sem_y):
    wid = lax.axis_index("s")
    base = wid * _CHUNK
    cp_t = pltpu.make_async_copy(table_hbm, table_v, sem_t)
    cp_y = pltpu.make_async_copy(y_hbm.at[pl.ds(base, _CHUNK)], y_v, sem_y)
    cp_t.start()
    cp_y.start()
    cp_t.wait()
    cp_y.wait()
    for i in range(_CHUNK // _L):
        idx = y_v[pl.ds(i * _L, _L)]
        out_v[pl.ds(i * _L, _L)] = plsc.load_gather(table_v, [idx])
    pltpu.sync_copy(out_v, out_hbm.at[pl.ds(base, _CHUNK)])


def kernel(y, table):
    run = pl.kernel(
        _lookup_body,
        out_type=jax.ShapeDtypeStruct((_N,), jnp.int32),
        mesh=plsc.VectorSubcoreMesh(
            core_axis_name="c", subcore_axis_name="s", num_cores=1
        ),
        compiler_params=pltpu.CompilerParams(needs_layout_passes=False),
        scratch_types=[
            pltpu.VMEM((_CHUNK,), jnp.int32),
            pltpu.VMEM((_CHUNK,), jnp.int32),
            pltpu.VMEM((_TABLE,), jnp.int32),
            pltpu.SemaphoreType.DMA,
            pltpu.SemaphoreType.DMA,
        ],
    )
    return run(y.astype(jnp.int32), table.astype(jnp.int32))


# speedup vs baseline: 1.0929x; 1.0025x over previous
import jax
import jax.numpy as jnp
from jax import lax
from jax.experimental import pallas as pl
from jax.experimental.pallas import tpu as pltpu
from jax.experimental.pallas import tpu_sc as plsc

_N = 16384
_TABLE = 10
_NS = 16
_L = 16
_CHUNK = _N // _NS


def _lookup_body(y_hbm, table_hbm, out_hbm, buf_v, table_v, sem_t, sem_y):
    wid = lax.axis_index("s")
    base = wid * _CHUNK
    cp_t = pltpu.make_async_copy(table_hbm, table_v, sem_t)
    cp_y = pltpu.make_async_copy(y_hbm.at[pl.ds(base, _CHUNK)], buf_v, sem_y)
    cp_t.start()
    cp_y.start()
    cp_t.wait()
    cp_y.wait()

    def step(i, _):
        idx = buf_v[pl.ds(i * _L, _L)]
        buf_v[pl.ds(i * _L, _L)] = plsc.load_gather(table_v, [idx])
        return ()

    lax.fori_loop(0, _CHUNK // _L, step, ())
    pltpu.sync_copy(buf_v, out_hbm.at[pl.ds(base, _CHUNK)])


def kernel(y, table):
    run = pl.kernel(
        _lookup_body,
        out_type=jax.ShapeDtypeStruct((_N,), jnp.int32),
        mesh=plsc.VectorSubcoreMesh(
            core_axis_name="c", subcore_axis_name="s", num_cores=1
        ),
        compiler_params=pltpu.CompilerParams(needs_layout_passes=False),
        scratch_types=[
            pltpu.VMEM((_CHUNK,), jnp.int32),
            pltpu.VMEM((_TABLE,), jnp.int32),
            pltpu.SemaphoreType.DMA,
            pltpu.SemaphoreType.DMA,
        ],
    )
    return run(y.astype(jnp.int32), table.astype(jnp.int32))
